# Initial kernel scaffold; baseline (speedup 1.0000x reference)
import jax
import jax.numpy as jnp
from jax.experimental import pallas as pl


def kernel(sp_embeddings, ft_embeddings, covariance, occupancy, slabels, clabels):
    raise NotImplementedError("write your pallas kernel here")


# speedup vs baseline: 7.9037x; 7.9037x over previous
import jax
import jax.numpy as jnp
from jax.experimental import pallas as pl
from jax.experimental.pallas import tpu as pltpu

_EPS = 0.001
_FT_INTER = 1.5
_SP_INTER = 0.2
_FT_REG_W = 0.1
_C = 256


def _p1_body(sp_ref, ft_ref, cov_ref, slab_ref, clab_ref, s_ref):
    i = pl.program_id(0)
    B = sp_ref.shape[0]
    slab = slab_ref[0]
    clab = clab_ref[0]
    seg = slab * 64 + clab
    col = jax.lax.broadcasted_iota(jnp.int32, (B, _C), 1)
    onehot = ((col == seg) & (slab < 4)).astype(jnp.float32)
    feat = jnp.concatenate(
        [sp_ref[...], ft_ref[...], cov_ref[...],
         jnp.ones((B, 1), jnp.float32), jnp.zeros((B, 2), jnp.float32)],
        axis=1)
    part = jax.lax.dot_general(onehot, feat, (((0,), (0,)), ((), ())),
                               preferred_element_type=jnp.float32)

    @pl.when(i == 0)
    def _():
        s_ref[...] = jnp.zeros_like(s_ref)

    s_ref[...] += part


def _p2_body(sp_ref, ft_ref, occ_ref, slab_ref, clab_ref, s_ref,
             loss_ref, acc_ref, colacc, segacc):
    j = pl.program_id(0)
    nb = pl.num_programs(0)
    B = sp_ref.shape[0]

    S = s_ref[...]
    cnt = S[:, 21:22]
    cntc = jnp.maximum(cnt, 1.0)
    inv = 1.0 / cntc
    valid = (cnt > 0.0).astype(jnp.float32)
    sp_c = S[:, 0:3] * inv
    ft_c = S[:, 3:19] * inv
    cov0 = jnp.maximum(S[:, 19:20] * inv, _EPS)
    cov1 = jnp.maximum(S[:, 20:21] * inv, _EPS)
    isp2 = 1.0 / (cov0 * cov0)
    ift2 = 1.0 / (cov1 * cov1)
    spc2 = jnp.sum(sp_c * sp_c, axis=1, keepdims=True)
    ftc2 = jnp.sum(ft_c * ft_c, axis=1, keepdims=True)
    acol = spc2 * isp2 + ftc2 * ift2
    logbc = jnp.log(cntc)
    V = jnp.concatenate(
        [-2.0 * sp_c * isp2, -2.0 * ft_c * ift2, isp2, ift2, acol,
         jnp.zeros((_C, 2), jnp.float32)], axis=1)
    W = jnp.concatenate(
        [sp_c, ft_c, logbc, isp2, ift2, jnp.zeros((_C, 3), jnp.float32)],
        axis=1)

    sp = sp_ref[...]
    ft = ft_ref[...]
    sp2 = jnp.sum(sp * sp, axis=1, keepdims=True)
    ft2 = jnp.sum(ft * ft, axis=1, keepdims=True)
    U = jnp.concatenate(
        [sp, ft, sp2, ft2, jnp.ones((B, 1), jnp.float32),
         jnp.zeros((B, 2), jnp.float32)], axis=1)

    q = jax.lax.dot_general(U, V, (((1,), (1,)), ((), ())),
                            preferred_element_type=jnp.float32)
    pvec = jnp.exp(-q)
    pc = jnp.clip(pvec, _EPS, 1.0 - _EPS)

    slab = slab_ref[0]
    clab = clab_ref[0]
    seg = slab * 64 + clab
    colid = jax.lax.broadcasted_iota(jnp.int32, (B, _C), 1)
    validr = jnp.reshape(valid, (1, _C))
    maskf = ((colid // 64 == slab) & (validr > 0.0)).astype(jnp.float32)
    bce_sp = maskf * (-jnp.log1p(-pc))
    posf = maskf * (pvec > 0.5).astype(jnp.float32)
    tf = ((colid == seg) & (validr > 0.0)).astype(jnp.float32)

    own = jax.lax.dot_general(tf, W, (((1,), (0,)), ((), ())),
                              preferred_element_type=jnp.float32)
    dsp = sp - own[:, 0:3]
    dft = ft - own[:, 3:19]
    dsp2 = jnp.sum(dsp * dsp, axis=1, keepdims=True)
    dft2 = jnp.sum(dft * dft, axis=1, keepdims=True)
    q_o = dsp2 * own[:, 20:21] + dft2 * own[:, 21:22]
    pv_o = jnp.exp(-q_o)
    pc_o = jnp.clip(pv_o, _EPS, 1.0 - _EPS)
    l_o = jnp.log(pc_o) - jnp.log1p(-pc_o)
    hft = jnp.clip(jnp.sqrt(dft2 + 1e-12) - 1.0, 0.0) ** 2
    hsp = dsp2 + 1e-12
    eocc = jnp.abs(occ_ref[...] - own[:, 19:20])
    pos_o = (pv_o > 0.5).astype(jnp.float32)
    h8 = jnp.concatenate(
        [hft, hsp, eocc, l_o, pos_o, jnp.zeros((B, 3), jnp.float32)],
        axis=1)
    hseg = jax.lax.dot_general(tf, h8, (((0,), (0,)), ((), ())),
                               preferred_element_type=jnp.float32)

    @pl.when(j == 0)
    def _():
        colacc[...] = jnp.zeros_like(colacc)
        segacc[...] = jnp.zeros_like(segacc)

    colacc[0:1, :] += jnp.sum(bce_sp, axis=0, keepdims=True)
    colacc[1:2, :] += jnp.sum(posf, axis=0, keepdims=True)
    segacc[...] += hseg

    @pl.when(j == nb - 1)
    def _():
        sa = segacc[...]
        bsum = jnp.reshape(colacc[0:1, :], (_C, 1))
        psum = jnp.reshape(colacc[1:2, :], (_C, 1))

        r = jax.lax.broadcasted_iota(jnp.int32, (_C, _C), 0)
        c = jax.lax.broadcasted_iota(jnp.int32, (_C, _C), 1)
        eyef = (r == c).astype(jnp.float32)
        pairm = ((r // 64 == c // 64).astype(jnp.float32) - eyef) \
            * valid * jnp.reshape(valid, (1, _C))
        gft = jax.lax.dot_general(ft_c, ft_c, (((1,), (1,)), ((), ())),
                                  preferred_element_type=jnp.float32)
        sqft = jnp.maximum(ftc2 + jnp.reshape(ftc2, (1, _C)) - 2.0 * gft, 0.0)
        dftm = jnp.sqrt(sqft + eyef)
        hft_row = jnp.sum(jnp.clip(2.0 * _FT_INTER - dftm, 0.0) ** 2 * pairm,
                          axis=1, keepdims=True)
        gsp = jax.lax.dot_general(sp_c, sp_c, (((1,), (1,)), ((), ())),
                                  preferred_element_type=jnp.float32)
        sqsp = jnp.maximum(spc2 + jnp.reshape(spc2, (1, _C)) - 2.0 * gsp, 0.0)
        dspm = jnp.sqrt(sqsp + eyef)
        hsp_row = jnp.sum(jnp.clip(2.0 * _SP_INTER - dspm, 0.0) ** 2 * pairm,
                          axis=1, keepdims=True)

        ftreg = jnp.sqrt(ftc2 + 1e-12) * valid
        qmat = jnp.concatenate(
            [valid, cnt, ftreg, bsum, psum,
             sa[:, 0:1] * inv * valid,
             sa[:, 1:2] * inv * valid,
             sa[:, 2:3] * inv * valid,
             sa[:, 3:4],
             sa[:, 4:5],
             hft_row, hsp_row,
             jnp.zeros((_C, 4), jnp.float32)], axis=1)
        clsind = (jax.lax.broadcasted_iota(jnp.int32, (_C, 4), 0) // 64
                  == jax.lax.broadcasted_iota(jnp.int32, (_C, 4), 1)
                  ).astype(jnp.float32)
        cls = jax.lax.dot_general(clsind, qmat, (((0,), (0,)), ((), ())),
                                  preferred_element_type=jnp.float32)

        cf = cls[:, 0:1]
        npts = cls[:, 1:2]
        vflag = (cf >= 2.0).astype(jnp.float32)
        pair_den = cf * (cf - 1.0)
        ft_loss = cls[:, 10:11] / pair_den + cls[:, 5:6] / cf \
            + _FT_REG_W * cls[:, 2:3] / cf
        sp_loss = cls[:, 11:12] / pair_den + cls[:, 6:7] / cf
        cov_loss = (cls[:, 3:4] - cls[:, 8:9]) / (npts * cf)
        occ_l = cls[:, 7:8] / cf
        inter_b = cls[:, 9:10]
        union_b = cls[:, 4:5] + npts - inter_b
        acc_c = inter_b / jnp.maximum(union_b, 1.0)
        loss_c = ft_loss + sp_loss + cov_loss + occ_l
        lsum = jnp.sum(jnp.where(vflag > 0.0, loss_c, 0.0))
        asum = jnp.sum(jnp.where(vflag > 0.0, acc_c, 0.0))
        vsum = jnp.maximum(jnp.sum(vflag), 1.0)
        loss_ref[...] = jnp.full((8, 128), lsum / vsum, jnp.float32)
        acc_ref[...] = jnp.full((8, 128), asum / vsum, jnp.float32)


def kernel(sp_embeddings, ft_embeddings, covariance, occupancy,
           slabels, clabels):
    n = sp_embeddings.shape[0]
    blk = 8
    for cand in range(2048, 7, -8):
        if n % cand == 0:
            blk = cand
            break
    nb = n // blk
    slabc = slabels.astype(jnp.int32).reshape(nb, blk, 1)
    clabc = clabels.astype(jnp.int32).reshape(nb, blk, 1)

    stats = pl.pallas_call(
        _p1_body,
        grid=(nb,),
        in_specs=[
            pl.BlockSpec((blk, 3), lambda i: (i, 0)),
            pl.BlockSpec((blk, 16), lambda i: (i, 0)),
            pl.BlockSpec((blk, 2), lambda i: (i, 0)),
            pl.BlockSpec((1, blk, 1), lambda i: (i, 0, 0)),
            pl.BlockSpec((1, blk, 1), lambda i: (i, 0, 0)),
        ],
        out_specs=pl.BlockSpec((_C, 24), lambda i: (0, 0)),
        out_shape=jax.ShapeDtypeStruct((_C, 24), jnp.float32),
    )(sp_embeddings, ft_embeddings, covariance, slabc, clabc)

    lossb, accb = pl.pallas_call(
        _p2_body,
        grid=(nb,),
        in_specs=[
            pl.BlockSpec((blk, 3), lambda i: (i, 0)),
            pl.BlockSpec((blk, 16), lambda i: (i, 0)),
            pl.BlockSpec((blk, 1), lambda i: (i, 0)),
            pl.BlockSpec((1, blk, 1), lambda i: (i, 0, 0)),
            pl.BlockSpec((1, blk, 1), lambda i: (i, 0, 0)),
            pl.BlockSpec((_C, 24), lambda i: (0, 0)),
        ],
        out_specs=[
            pl.BlockSpec((8, 128), lambda i: (0, 0)),
            pl.BlockSpec((8, 128), lambda i: (0, 0)),
        ],
        out_shape=[
            jax.ShapeDtypeStruct((8, 128), jnp.float32),
            jax.ShapeDtypeStruct((8, 128), jnp.float32),
        ],
        scratch_shapes=[
            pltpu.VMEM((8, _C), jnp.float32),
            pltpu.VMEM((_C, 8), jnp.float32),
        ],
    )(sp_embeddings, ft_embeddings, occupancy, slabc, clabc, stats)

    return lossb[0, 0], accb[0, 0]


# speedup vs baseline: 8.9375x; 1.1308x over previous
import jax
import jax.numpy as jnp
from jax.experimental import pallas as pl
from jax.experimental.pallas import tpu as pltpu

_EPS = 0.001
_FT_INTER = 1.5
_SP_INTER = 0.2
_FT_REG_W = 0.1
_C = 256


def _p1_body(sp_ref, ft_ref, cov_ref, slab_ref, clab_ref, s_ref):
    i = pl.program_id(0)
    B = sp_ref.shape[0]
    slab = slab_ref[0]
    clab = clab_ref[0]
    seg = slab * 64 + clab
    col = jax.lax.broadcasted_iota(jnp.int32, (B, _C), 1)
    onehot = (col == seg).astype(jnp.float32)
    feat = jnp.concatenate(
        [sp_ref[...], ft_ref[...], cov_ref[...],
         jnp.ones((B, 1), jnp.float32), jnp.zeros((B, 2), jnp.float32)],
        axis=1)
    part = jax.lax.dot_general(onehot, feat, (((0,), (0,)), ((), ())),
                               preferred_element_type=jnp.float32)

    @pl.when(i == 0)
    def _():
        s_ref[...] = jnp.zeros_like(s_ref)

    s_ref[...] += part


def _p2_body(sp_ref, ft_ref, occ_ref, slab_ref, clab_ref, s_ref,
             loss_ref, acc_ref, colacc, segacc):
    j = pl.program_id(0)
    nb = pl.num_programs(0)
    B = sp_ref.shape[0]

    S = s_ref[...]
    cnt = S[:, 21:22]
    cntc = jnp.maximum(cnt, 1.0)
    inv = 1.0 / cntc
    sp_c = S[:, 0:3] * inv
    ft_c = S[:, 3:19] * inv
    cov0 = jnp.maximum(S[:, 19:20] * inv, _EPS)
    cov1 = jnp.maximum(S[:, 20:21] * inv, _EPS)
    isp2 = 1.0 / (cov0 * cov0)
    ift2 = 1.0 / (cov1 * cov1)
    spc2 = jnp.sum(sp_c * sp_c, axis=1, keepdims=True)
    ftc2 = jnp.sum(ft_c * ft_c, axis=1, keepdims=True)
    acol = spc2 * isp2 + ftc2 * ift2
    logbc = jnp.log(cntc)
    V = jnp.concatenate(
        [-2.0 * sp_c * isp2, -2.0 * ft_c * ift2, isp2, ift2, acol,
         jnp.zeros((_C, 2), jnp.float32)], axis=1)
    W = jnp.concatenate(
        [sp_c, ft_c, logbc, isp2, ift2, jnp.zeros((_C, 3), jnp.float32)],
        axis=1)

    sp = sp_ref[...]
    ft = ft_ref[...]
    sp2 = jnp.sum(sp * sp, axis=1, keepdims=True)
    ft2 = jnp.sum(ft * ft, axis=1, keepdims=True)
    U = jnp.concatenate(
        [sp, ft, sp2, ft2, jnp.ones((B, 1), jnp.float32),
         jnp.zeros((B, 2), jnp.float32)], axis=1)

    q = jax.lax.dot_general(U, V, (((1,), (1,)), ((), ())),
                            preferred_element_type=jnp.float32)
    pvec = jnp.exp(-q)
    pc = jnp.clip(pvec, _EPS, 1.0 - _EPS)
    xb = -jnp.log1p(-pc)
    xp = (pvec > 0.5).astype(jnp.float32)

    slab = slab_ref[0]
    clab = clab_ref[0]
    seg = slab * 64 + clab
    colid = jax.lax.broadcasted_iota(jnp.int32, (B, _C), 1)
    tf = (colid == seg).astype(jnp.float32)
    clsoh = (jax.lax.broadcasted_iota(jnp.int32, (B, 8), 1)
             == slab).astype(jnp.float32)

    own = jax.lax.dot_general(tf, W, (((1,), (0,)), ((), ())),
                              preferred_element_type=jnp.float32)
    dsp = sp - own[:, 0:3]
    dft = ft - own[:, 3:19]
    dsp2 = jnp.sum(dsp * dsp, axis=1, keepdims=True)
    dft2 = jnp.sum(dft * dft, axis=1, keepdims=True)
    q_o = dsp2 * own[:, 20:21] + dft2 * own[:, 21:22]
    pv_o = jnp.exp(-q_o)
    pc_o = jnp.clip(pv_o, _EPS, 1.0 - _EPS)
    l_o = jnp.log(pc_o) - jnp.log1p(-pc_o)
    hft = jnp.clip(jnp.sqrt(dft2 + 1e-12) - 1.0, 0.0) ** 2
    hsp = dsp2 + 1e-12
    eocc = jnp.abs(occ_ref[...] - own[:, 19:20])
    pos_o = (pv_o > 0.5).astype(jnp.float32)
    h8 = jnp.concatenate(
        [hft, hsp, eocc, l_o, pos_o, jnp.zeros((B, 3), jnp.float32)],
        axis=1)
    hseg = jax.lax.dot_general(tf, h8, (((0,), (0,)), ((), ())),
                               preferred_element_type=jnp.float32)
    mb = jax.lax.dot_general(clsoh, xb, (((0,), (0,)), ((), ())),
                             preferred_element_type=jnp.float32)
    mp = jax.lax.dot_general(clsoh, xp, (((0,), (0,)), ((), ())),
                             preferred_element_type=jnp.float32)

    @pl.when(j == 0)
    def _():
        colacc[...] = jnp.zeros_like(colacc)
        segacc[...] = jnp.zeros_like(segacc)

    colacc[0:8, :] += mb
    colacc[8:16, :] += mp
    segacc[...] += hseg

    @pl.when(j == nb - 1)
    def _():
        sa = segacc[...]
        valid = (cnt > 0.0).astype(jnp.float32)
        validr = jnp.reshape(valid, (1, _C))
        rowcls = jax.lax.broadcasted_iota(jnp.int32, (8, _C), 0)
        colcls8 = jax.lax.broadcasted_iota(jnp.int32, (8, _C), 1) // 64
        selm = (rowcls == colcls8).astype(jnp.float32) * validr
        bcls = jnp.sum(colacc[0:8, :] * selm, axis=1, keepdims=True)
        pcls = jnp.sum(colacc[8:16, :] * selm, axis=1, keepdims=True)

        r = jax.lax.broadcasted_iota(jnp.int32, (_C, _C), 0)
        c = jax.lax.broadcasted_iota(jnp.int32, (_C, _C), 1)
        eyef = (r == c).astype(jnp.float32)
        pairm = ((r // 64 == c // 64).astype(jnp.float32) - eyef) \
            * valid * validr
        gft = jax.lax.dot_general(ft_c, ft_c, (((1,), (1,)), ((), ())),
                                  preferred_element_type=jnp.float32)
        sqft = jnp.maximum(ftc2 + jnp.reshape(ftc2, (1, _C)) - 2.0 * gft, 0.0)
        dftm = jnp.sqrt(sqft + eyef)
        hft_row = jnp.sum(jnp.clip(2.0 * _FT_INTER - dftm, 0.0) ** 2 * pairm,
                          axis=1, keepdims=True)
        gsp = jax.lax.dot_general(sp_c, sp_c, (((1,), (1,)), ((), ())),
                                  preferred_element_type=jnp.float32)
        sqsp = jnp.maximum(spc2 + jnp.reshape(spc2, (1, _C)) - 2.0 * gsp, 0.0)
        dspm = jnp.sqrt(sqsp + eyef)
        hsp_row = jnp.sum(jnp.clip(2.0 * _SP_INTER - dspm, 0.0) ** 2 * pairm,
                          axis=1, keepdims=True)

        ftreg = jnp.sqrt(ftc2 + 1e-12) * valid
        qmat = jnp.concatenate(
            [valid, cnt, ftreg,
             sa[:, 0:1] * inv * valid,
             sa[:, 1:2] * inv * valid,
             sa[:, 2:3] * inv * valid,
             sa[:, 3:4] * valid,
             sa[:, 4:5] * valid,
             hft_row, hsp_row,
             jnp.zeros((_C, 6), jnp.float32)], axis=1)
        clsind = (jax.lax.broadcasted_iota(jnp.int32, (_C, 4), 0) // 64
                  == jax.lax.broadcasted_iota(jnp.int32, (_C, 4), 1)
                  ).astype(jnp.float32)
        cls = jax.lax.dot_general(clsind, qmat, (((0,), (0,)), ((), ())),
                                  preferred_element_type=jnp.float32)

        cf = cls[:, 0:1]
        npts = cls[:, 1:2]
        bsum = bcls[0:4]
        psum = pcls[0:4]
        vflag = (cf >= 2.0).astype(jnp.float32)
        pair_den = cf * (cf - 1.0)
        ft_loss = cls[:, 8:9] / pair_den + cls[:, 3:4] / cf \
            + _FT_REG_W * cls[:, 2:3] / cf
        sp_loss = cls[:, 9:10] / pair_den + cls[:, 4:5] / cf
        cov_loss = (bsum - cls[:, 6:7]) / (npts * cf)
        occ_l = cls[:, 5:6] / cf
        inter_b = cls[:, 7:8]
        union_b = psum + npts - inter_b
        acc_c = inter_b / jnp.maximum(union_b, 1.0)
        loss_c = ft_loss + sp_loss + cov_loss + occ_l
        lsum = jnp.sum(jnp.where(vflag > 0.0, loss_c, 0.0))
        asum = jnp.sum(jnp.where(vflag > 0.0, acc_c, 0.0))
        vsum = jnp.maximum(jnp.sum(vflag), 1.0)
        loss_ref[...] = jnp.full((8, 128), lsum / vsum, jnp.float32)
        acc_ref[...] = jnp.full((8, 128), asum / vsum, jnp.float32)


def kernel(sp_embeddings, ft_embeddings, covariance, occupancy,
           slabels, clabels):
    n = sp_embeddings.shape[0]
    blk = 8
    for cand in range(2048, 7, -8):
        if n % cand == 0:
            blk = cand
            break
    nb = n // blk
    slabc = slabels.astype(jnp.int32).reshape(nb, blk, 1)
    clabc = clabels.astype(jnp.int32).reshape(nb, blk, 1)

    stats = pl.pallas_call(
        _p1_body,
        grid=(nb,),
        in_specs=[
            pl.BlockSpec((blk, 3), lambda i: (i, 0)),
            pl.BlockSpec((blk, 16), lambda i: (i, 0)),
            pl.BlockSpec((blk, 2), lambda i: (i, 0)),
            pl.BlockSpec((1, blk, 1), lambda i: (i, 0, 0)),
            pl.BlockSpec((1, blk, 1), lambda i: (i, 0, 0)),
        ],
        out_specs=pl.BlockSpec((_C, 24), lambda i: (0, 0)),
        out_shape=jax.ShapeDtypeStruct((_C, 24), jnp.float32),
    )(sp_embeddings, ft_embeddings, covariance, slabc, clabc)

    lossb, accb = pl.pallas_call(
        _p2_body,
        grid=(nb,),
        in_specs=[
            pl.BlockSpec((blk, 3), lambda i: (i, 0)),
            pl.BlockSpec((blk, 16), lambda i: (i, 0)),
            pl.BlockSpec((blk, 1), lambda i: (i, 0)),
            pl.BlockSpec((1, blk, 1), lambda i: (i, 0, 0)),
            pl.BlockSpec((1, blk, 1), lambda i: (i, 0, 0)),
            pl.BlockSpec((_C, 24), lambda i: (0, 0)),
        ],
        out_specs=[
            pl.BlockSpec((8, 128), lambda i: (0, 0)),
            pl.BlockSpec((8, 128), lambda i: (0, 0)),
        ],
        out_shape=[
            jax.ShapeDtypeStruct((8, 128), jnp.float32),
            jax.ShapeDtypeStruct((8, 128), jnp.float32),
        ],
        scratch_shapes=[
            pltpu.VMEM((16, _C), jnp.float32),
            pltpu.VMEM((_C, 8), jnp.float32),
        ],
    )(sp_embeddings, ft_embeddings, occupancy, slabc, clabc, stats)

    return lossb[0, 0], accb[0, 0]


# speedup vs baseline: 9.1424x; 1.0229x over previous
import jax
import jax.numpy as jnp
from jax.experimental import pallas as pl
from jax.experimental.pallas import tpu as pltpu

_EPS = 0.001
_FT_INTER = 1.5
_SP_INTER = 0.2
_FT_REG_W = 0.1
_C = 256


def _p1_body(sp_ref, ft_ref, cov_ref, slab_ref, clab_ref, p_ref, s_ref):
    i = pl.program_id(0)
    nb = pl.num_programs(0)
    B = sp_ref.shape[0]
    slab = slab_ref[0]
    clab = clab_ref[0]
    seg = slab * 64 + clab
    col = jax.lax.broadcasted_iota(jnp.int32, (B, _C), 1)
    onehot = (col == seg).astype(jnp.float32)
    feat = jnp.concatenate(
        [sp_ref[...], ft_ref[...], cov_ref[...],
         jnp.ones((B, 1), jnp.float32), jnp.zeros((B, 2), jnp.float32)],
        axis=1)
    part = jax.lax.dot_general(onehot, feat, (((0,), (0,)), ((), ())),
                               preferred_element_type=jnp.float32)

    @pl.when(i == 0)
    def _():
        s_ref[...] = jnp.zeros_like(s_ref)

    s_ref[...] += part

    @pl.when(i == nb - 1)
    def _():
        S = s_ref[...]
        cnt = S[:, 21:22]
        cntc = jnp.maximum(cnt, 1.0)
        inv = 1.0 / cntc
        valid = (cnt > 0.0).astype(jnp.float32)
        sp_c = S[:, 0:3] * inv
        ft_c = S[:, 3:19] * inv
        cov0 = jnp.maximum(S[:, 19:20] * inv, _EPS)
        cov1 = jnp.maximum(S[:, 20:21] * inv, _EPS)
        isp2 = 1.0 / (cov0 * cov0)
        ift2 = 1.0 / (cov1 * cov1)
        spc2 = jnp.sum(sp_c * sp_c, axis=1, keepdims=True)
        ftc2 = jnp.sum(ft_c * ft_c, axis=1, keepdims=True)
        acol = spc2 * isp2 + ftc2 * ift2
        logbc = jnp.log(cntc)
        p_ref[...] = jnp.concatenate(
            [-2.0 * sp_c * isp2, -2.0 * ft_c * ift2, isp2, ift2, acol,
             jnp.zeros((_C, 2), jnp.float32),
             sp_c, ft_c, logbc, isp2, ift2,
             jnp.zeros((_C, 2), jnp.float32),
             cnt, inv, valid, spc2, ftc2,
             jnp.zeros((_C, 3), jnp.float32)], axis=1)


def _p2_body(sp_ref, ft_ref, occ_ref, slab_ref, clab_ref, s_ref,
             loss_ref, acc_ref, colacc, segacc):
    j = pl.program_id(0)
    nb = pl.num_programs(0)
    B = sp_ref.shape[0]

    S = s_ref[...]
    V = S[:, 0:24]
    W = S[:, 24:48]
    cnt = S[:, 48:49]
    inv = S[:, 49:50]
    sp_c = W[:, 0:3]
    ft_c = W[:, 3:19]
    spc2 = S[:, 51:52]
    ftc2 = S[:, 52:53]

    sp = sp_ref[...]
    ft = ft_ref[...]
    sp2 = jnp.sum(sp * sp, axis=1, keepdims=True)
    ft2 = jnp.sum(ft * ft, axis=1, keepdims=True)
    U = jnp.concatenate(
        [sp, ft, sp2, ft2, jnp.ones((B, 1), jnp.float32),
         jnp.zeros((B, 2), jnp.float32)], axis=1)

    q = jax.lax.dot_general(U, V, (((1,), (1,)), ((), ())),
                            preferred_element_type=jnp.float32)
    pvec = jnp.exp(-q)
    pc = jnp.clip(pvec, _EPS, 1.0 - _EPS)
    xb = -jnp.log1p(-pc)
    xp = (pvec > 0.5).astype(jnp.float32)

    slab = slab_ref[0]
    clab = clab_ref[0]
    seg = slab * 64 + clab
    colid = jax.lax.broadcasted_iota(jnp.int32, (B, _C), 1)
    tf = (colid == seg).astype(jnp.float32)
    clsoh = (jax.lax.broadcasted_iota(jnp.int32, (B, 8), 1)
             == slab).astype(jnp.float32)

    own = jax.lax.dot_general(tf, W, (((1,), (0,)), ((), ())),
                              preferred_element_type=jnp.float32)
    dsp = sp - own[:, 0:3]
    dft = ft - own[:, 3:19]
    dsp2 = jnp.sum(dsp * dsp, axis=1, keepdims=True)
    dft2 = jnp.sum(dft * dft, axis=1, keepdims=True)
    q_o = dsp2 * own[:, 20:21] + dft2 * own[:, 21:22]
    pv_o = jnp.exp(-q_o)
    pc_o = jnp.clip(pv_o, _EPS, 1.0 - _EPS)
    l_o = jnp.log(pc_o) - jnp.log1p(-pc_o)
    hft = jnp.clip(jnp.sqrt(dft2 + 1e-12) - 1.0, 0.0) ** 2
    hsp = dsp2 + 1e-12
    eocc = jnp.abs(occ_ref[...] - own[:, 19:20])
    pos_o = (pv_o > 0.5).astype(jnp.float32)
    h8 = jnp.concatenate(
        [hft, hsp, eocc, l_o, pos_o, jnp.zeros((B, 3), jnp.float32)],
        axis=1)
    hseg = jax.lax.dot_general(tf, h8, (((0,), (0,)), ((), ())),
                               preferred_element_type=jnp.float32)
    mb = jax.lax.dot_general(clsoh, xb, (((0,), (0,)), ((), ())),
                             preferred_element_type=jnp.float32)
    mp = jax.lax.dot_general(clsoh, xp, (((0,), (0,)), ((), ())),
                             preferred_element_type=jnp.float32)

    @pl.when(j == 0)
    def _():
        colacc[...] = jnp.zeros_like(colacc)
        segacc[...] = jnp.zeros_like(segacc)

    colacc[0:8, :] += mb
    colacc[8:16, :] += mp
    segacc[...] += hseg

    @pl.when(j == nb - 1)
    def _():
        sa = segacc[...]
        valid = (cnt > 0.0).astype(jnp.float32)
        validr = jnp.reshape(valid, (1, _C))
        rowcls = jax.lax.broadcasted_iota(jnp.int32, (8, _C), 0)
        colcls8 = jax.lax.broadcasted_iota(jnp.int32, (8, _C), 1) // 64
        selm = (rowcls == colcls8).astype(jnp.float32) * validr
        bcls = jnp.sum(colacc[0:8, :] * selm, axis=1, keepdims=True)
        pcls = jnp.sum(colacc[8:16, :] * selm, axis=1, keepdims=True)

        r = jax.lax.broadcasted_iota(jnp.int32, (_C, _C), 0)
        c = jax.lax.broadcasted_iota(jnp.int32, (_C, _C), 1)
        eyef = (r == c).astype(jnp.float32)
        pairm = ((r // 64 == c // 64).astype(jnp.float32) - eyef) \
            * valid * validr
        gft = jax.lax.dot_general(ft_c, ft_c, (((1,), (1,)), ((), ())),
                                  preferred_element_type=jnp.float32)
        sqft = jnp.maximum(ftc2 + jnp.reshape(ftc2, (1, _C)) - 2.0 * gft, 0.0)
        dftm = jnp.sqrt(sqft + eyef)
        hft_row = jnp.sum(jnp.clip(2.0 * _FT_INTER - dftm, 0.0) ** 2 * pairm,
                          axis=1, keepdims=True)
        gsp = jax.lax.dot_general(sp_c, sp_c, (((1,), (1,)), ((), ())),
                                  preferred_element_type=jnp.float32)
        sqsp = jnp.maximum(spc2 + jnp.reshape(spc2, (1, _C)) - 2.0 * gsp, 0.0)
        dspm = jnp.sqrt(sqsp + eyef)
        hsp_row = jnp.sum(jnp.clip(2.0 * _SP_INTER - dspm, 0.0) ** 2 * pairm,
                          axis=1, keepdims=True)

        ftreg = jnp.sqrt(ftc2 + 1e-12) * valid
        qmat = jnp.concatenate(
            [valid, cnt, ftreg,
             sa[:, 0:1] * inv * valid,
             sa[:, 1:2] * inv * valid,
             sa[:, 2:3] * inv * valid,
             sa[:, 3:4] * valid,
             sa[:, 4:5] * valid,
             hft_row, hsp_row,
             jnp.zeros((_C, 6), jnp.float32)], axis=1)
        clsind = (jax.lax.broadcasted_iota(jnp.int32, (_C, 4), 0) // 64
                  == jax.lax.broadcasted_iota(jnp.int32, (_C, 4), 1)
                  ).astype(jnp.float32)
        cls = jax.lax.dot_general(clsind, qmat, (((0,), (0,)), ((), ())),
                                  preferred_element_type=jnp.float32)

        cf = cls[:, 0:1]
        npts = cls[:, 1:2]
        bsum = bcls[0:4]
        psum = pcls[0:4]
        vflag = (cf >= 2.0).astype(jnp.float32)
        pair_den = cf * (cf - 1.0)
        ft_loss = cls[:, 8:9] / pair_den + cls[:, 3:4] / cf \
            + _FT_REG_W * cls[:, 2:3] / cf
        sp_loss = cls[:, 9:10] / pair_den + cls[:, 4:5] / cf
        cov_loss = (bsum - cls[:, 6:7]) / (npts * cf)
        occ_l = cls[:, 5:6] / cf
        inter_b = cls[:, 7:8]
        union_b = psum + npts - inter_b
        acc_c = inter_b / jnp.maximum(union_b, 1.0)
        loss_c = ft_loss + sp_loss + cov_loss + occ_l
        lsum = jnp.sum(jnp.where(vflag > 0.0, loss_c, 0.0))
        asum = jnp.sum(jnp.where(vflag > 0.0, acc_c, 0.0))
        vsum = jnp.maximum(jnp.sum(vflag), 1.0)
        loss_ref[...] = jnp.full((8, 128), lsum / vsum, jnp.float32)
        acc_ref[...] = jnp.full((8, 128), asum / vsum, jnp.float32)


def kernel(sp_embeddings, ft_embeddings, covariance, occupancy,
           slabels, clabels):
    n = sp_embeddings.shape[0]
    blk = 8
    for cand in range(2048, 7, -8):
        if n % cand == 0:
            blk = cand
            break
    nb = n // blk
    slabc = slabels.astype(jnp.int32).reshape(nb, blk, 1)
    clabc = clabels.astype(jnp.int32).reshape(nb, blk, 1)

    stats = pl.pallas_call(
        _p1_body,
        grid=(nb,),
        in_specs=[
            pl.BlockSpec((blk, 3), lambda i: (i, 0)),
            pl.BlockSpec((blk, 16), lambda i: (i, 0)),
            pl.BlockSpec((blk, 2), lambda i: (i, 0)),
            pl.BlockSpec((1, blk, 1), lambda i: (i, 0, 0)),
            pl.BlockSpec((1, blk, 1), lambda i: (i, 0, 0)),
        ],
        out_specs=pl.BlockSpec((_C, 56), lambda i: (0, 0)),
        out_shape=jax.ShapeDtypeStruct((_C, 56), jnp.float32),
        scratch_shapes=[pltpu.VMEM((_C, 24), jnp.float32)],
    )(sp_embeddings, ft_embeddings, covariance, slabc, clabc)

    lossb, accb = pl.pallas_call(
        _p2_body,
        grid=(nb,),
        in_specs=[
            pl.BlockSpec((blk, 3), lambda i: (i, 0)),
            pl.BlockSpec((blk, 16), lambda i: (i, 0)),
            pl.BlockSpec((blk, 1), lambda i: (i, 0)),
            pl.BlockSpec((1, blk, 1), lambda i: (i, 0, 0)),
            pl.BlockSpec((1, blk, 1), lambda i: (i, 0, 0)),
            pl.BlockSpec((_C, 56), lambda i: (0, 0)),
        ],
        out_specs=[
            pl.BlockSpec((8, 128), lambda i: (0, 0)),
            pl.BlockSpec((8, 128), lambda i: (0, 0)),
        ],
        out_shape=[
            jax.ShapeDtypeStruct((8, 128), jnp.float32),
            jax.ShapeDtypeStruct((8, 128), jnp.float32),
        ],
        scratch_shapes=[
            pltpu.VMEM((16, _C), jnp.float32),
            pltpu.VMEM((_C, 8), jnp.float32),
        ],
    )(sp_embeddings, ft_embeddings, occupancy, slabc, clabc, stats)

    return lossb[0, 0], accb[0, 0]
